# baseline (speedup 1.0000x reference)
import jax
import jax.numpy as jnp
from jax import lax
from jax.experimental import pallas as pl
from jax.experimental.pallas import tpu as pltpu

B, C, H, W = 256, 256, 14, 14
HW = H * W
MAX_ID = 50000


def _fused_mask_multiply(indices, tbl_lines, qsel, x_planes, p):
    grid = (HW // p,)

    def body(idx_ref, x_ref, q_ref, tbl_ref, o_ref, m32, mask_v, sem):
        step = pl.program_id(0)

        @pl.when(step == 0)
        def _():
            def issue(k, carry):
                line = idx_ref[k] // 16
                pltpu.make_async_copy(
                    tbl_ref.at[pl.ds(line, 1), :],
                    m32.at[pl.ds(k, 1), :],
                    sem,
                ).start()
                return carry

            lax.fori_loop(0, B, issue, 0, unroll=8)

            def drain(k, carry):
                pltpu.make_async_copy(
                    tbl_ref.at[pl.ds(0, 1), :],
                    m32.at[pl.ds(k, 1), :],
                    sem,
                ).wait()
                return carry

            lax.fori_loop(0, B, drain, 0, unroll=8)

            m = m32[...]
            q = q_ref[...]
            m_sel = m[:, 0:8]
            for s in range(1, 16):
                m_sel = jnp.where(q == s, m[:, 8 * s : 8 * s + 8], m_sel)
            pieces = [(m_sel >> j) & 1 for j in range(32)]
            interl = jnp.concatenate(pieces, axis=1).astype(jnp.float32)
            li = lax.broadcasted_iota(jnp.int32, (C, C), 0)
            ci = lax.broadcasted_iota(jnp.int32, (C, C), 1)
            perm = (ci == 32 * (li % 8) + li // 8).astype(jnp.float32)
            mask_v[...] = jnp.dot(interl, perm,
                                  preferred_element_type=jnp.float32)

        o_ref[...] = x_ref[...] * mask_v[...][None, :, :]

    grid_spec = pltpu.PrefetchScalarGridSpec(
        num_scalar_prefetch=1,
        grid=grid,
        in_specs=[
            pl.BlockSpec((p, B, C), lambda i, idx: (i, 0, 0)),
            pl.BlockSpec((B, 1), lambda i, idx: (0, 0)),
            pl.BlockSpec(memory_space=pl.ANY),
        ],
        out_specs=pl.BlockSpec((p, B, C), lambda i, idx: (i, 0, 0)),
        scratch_shapes=[
            pltpu.VMEM((B, 128), jnp.uint32),
            pltpu.VMEM((B, C), jnp.float32),
            pltpu.SemaphoreType.DMA,
        ],
    )
    return pl.pallas_call(
        body,
        grid_spec=grid_spec,
        out_shape=jax.ShapeDtypeStruct((HW, B, C), jnp.float32),
    )(indices, x_planes, qsel, tbl_lines)


def kernel(X, indices, mask_table):
    bits = mask_table.reshape(MAX_ID // 16, 128, 32).astype(jnp.uint32)
    tbl_lines = jnp.sum(
        bits << jnp.arange(32, dtype=jnp.uint32), axis=-1, dtype=jnp.uint32
    )
    qsel = (indices % 16).astype(jnp.int32).reshape(B, 1)
    x_planes = jnp.transpose(X, (2, 3, 0, 1)).reshape(HW, B, C)
    out = _fused_mask_multiply(indices, tbl_lines, qsel, x_planes, p=49)
    return jnp.transpose(out.reshape(H, W, B, C), (2, 3, 0, 1))


# speedup vs baseline: 5.5457x; 5.5457x over previous
import jax
import jax.numpy as jnp
from jax import lax
from jax.experimental import pallas as pl
from jax.experimental.pallas import tpu as pltpu

B, C, H, W = 256, 256, 14, 14
HW = H * W
MAX_ID = 50000
NS = B * 8


def _fused_mask_multiply(indices, tbl_i8, rsel, x_planes, p):
    grid = (HW // p,)

    def body(idx_ref, x_ref, r_ref, tbl_ref, o_ref, slabs, mask_v, sem):
        step = pl.program_id(0)

        @pl.when(step == 0)
        def _():
            def issue(k, carry):
                base = (idx_ref[k] // 8) * 8
                pltpu.make_async_copy(
                    tbl_ref.at[pl.ds(base, 8), :],
                    slabs.at[pl.ds(k * 8, 8), :],
                    sem,
                ).start()
                return carry

            lax.fori_loop(0, B, issue, 0, unroll=8)

            def drain(k, carry):
                pltpu.make_async_copy(
                    tbl_ref.at[pl.ds(0, 8), :],
                    slabs.at[pl.ds(k * 8, 8), :],
                    sem,
                ).wait()
                return carry

            lax.fori_loop(0, B, drain, 0, unroll=8)

            s = slabs[...].astype(jnp.float32)
            r = r_ref[...]
            ki = lax.broadcasted_iota(jnp.int32, (B, NS), 0)
            ji = lax.broadcasted_iota(jnp.int32, (B, NS), 1)
            sel = (ji == 8 * ki + r).astype(jnp.float32)
            mask_v[...] = jnp.dot(sel, s, preferred_element_type=jnp.float32)

        o_ref[...] = x_ref[...] * mask_v[...][None, :, :]

    grid_spec = pltpu.PrefetchScalarGridSpec(
        num_scalar_prefetch=1,
        grid=grid,
        in_specs=[
            pl.BlockSpec((p, B, C), lambda i, idx: (i, 0, 0)),
            pl.BlockSpec((B, 1), lambda i, idx: (0, 0)),
            pl.BlockSpec(memory_space=pl.ANY),
        ],
        out_specs=pl.BlockSpec((p, B, C), lambda i, idx: (i, 0, 0)),
        scratch_shapes=[
            pltpu.VMEM((NS, C), jnp.int4),
            pltpu.VMEM((B, C), jnp.float32),
            pltpu.SemaphoreType.DMA,
        ],
    )
    return pl.pallas_call(
        body,
        grid_spec=grid_spec,
        out_shape=jax.ShapeDtypeStruct((HW, B, C), jnp.float32),
    )(indices, x_planes, rsel, tbl_i8)


def kernel(X, indices, mask_table):
    tbl_i8 = mask_table.astype(jnp.int4)
    rsel = (indices % 8).astype(jnp.int32).reshape(B, 1)
    x_planes = jnp.transpose(X, (2, 3, 0, 1)).reshape(HW, B, C)
    out = _fused_mask_multiply(indices, tbl_i8, rsel, x_planes, p=49)
    return jnp.transpose(out.reshape(H, W, B, C), (2, 3, 0, 1))
